# Initial kernel scaffold; baseline (speedup 1.0000x reference)
import jax
import jax.numpy as jnp
from jax.experimental import pallas as pl


def kernel(token_ids, weights):
    raise NotImplementedError("write your pallas kernel here")


# speedup vs baseline: 1.8457x; 1.8457x over previous
import functools

import jax
import jax.numpy as jnp
from jax import lax
from jax.experimental import pallas as pl
from jax.experimental.pallas import tpu as pltpu
from jax.experimental.pallas import tpu_sc as plsc

NUM_CORES = 2
NUM_SUBCORES = 16
NW = NUM_CORES * NUM_SUBCORES

SUB = 128
CHUNK = 1024
N_SUB = CHUNK // SUB


@functools.cache
def _build(B, V, D):
    assert B % (NW * CHUNK) == 0
    b_per_w = B // NW
    n_chunks = b_per_w // CHUNK
    mesh = plsc.VectorSubcoreMesh(core_axis_name="c", subcore_axis_name="s")

    @functools.partial(
        pl.kernel,
        mesh=mesh,
        out_type=jax.ShapeDtypeStruct((B, D), jnp.float32),
        scratch_types=[
            pltpu.VMEM((N_SUB, SUB), jnp.int32),
            pltpu.VMEM((CHUNK, D), jnp.float32),
            pltpu.SemaphoreType.DMA,
        ],
        compiler_params=pltpu.CompilerParams(use_tc_tiling_on_sc=False),
    )
    def gather_kernel(ids_hbm, table_hbm, out_hbm, idx_v, rows_v, sem):
        wid = lax.axis_index("s") * NUM_CORES + lax.axis_index("c")
        base = wid * b_per_w

        def chunk_body(c, carry):
            off = base + c * CHUNK
            idx_row = pl.multiple_of(off // SUB, 8)
            pltpu.sync_copy(ids_hbm.at[pl.ds(idx_row, N_SUB)], idx_v)
            copies = [
                pltpu.async_copy(
                    table_hbm.at[idx_v.at[j]],
                    rows_v.at[pl.ds(j * SUB, SUB)],
                    sem,
                )
                for j in range(N_SUB)
            ]
            for cp in copies:
                cp.wait()
            pltpu.sync_copy(rows_v, out_hbm.at[pl.ds(off, CHUNK)])
            return carry

        lax.fori_loop(0, n_chunks, chunk_body, 0)

    return gather_kernel


def kernel(token_ids, weights):
    B0, S = token_ids.shape
    V, D = weights.shape
    B = B0 * S
    ids = token_ids.reshape(B // SUB, SUB).astype(jnp.int32)
    out = _build(B, V, D)(ids, weights)
    return out.reshape(B0, S, D)


# speedup vs baseline: 1.8767x; 1.0168x over previous
import functools

import jax
import jax.numpy as jnp
from jax import lax
from jax.experimental import pallas as pl
from jax.experimental.pallas import tpu as pltpu
from jax.experimental.pallas import tpu_sc as plsc

NUM_CORES = 2
NUM_SUBCORES = 16
NW = NUM_CORES * NUM_SUBCORES

SUB = 128
G = 512
N_SUB = G // SUB
NBUF = 3


@functools.cache
def _build(B, V, D):
    assert B % (NW * G) == 0
    b_per_w = B // NW
    n_gran = b_per_w // G
    mesh = plsc.VectorSubcoreMesh(core_axis_name="c", subcore_axis_name="s")

    @functools.partial(
        pl.kernel,
        mesh=mesh,
        out_type=jax.ShapeDtypeStruct((B, D), jnp.float32),
        scratch_types=[
            pltpu.VMEM((NBUF, N_SUB, SUB), jnp.int32),
            pltpu.VMEM((NBUF, G, D), jnp.float32),
            pltpu.SemaphoreType.DMA,
            pltpu.SemaphoreType.DMA,
            pltpu.SemaphoreType.DMA,
        ],
        compiler_params=pltpu.CompilerParams(use_tc_tiling_on_sc=False),
    )
    def gather_kernel(ids_hbm, table_hbm, out_hbm, idx_v, rows_v, sem_i,
                      sem_g, sem_w):
        wid = lax.axis_index("s") * NUM_CORES + lax.axis_index("c")
        base = wid * b_per_w
        g0 = wid * n_gran

        def fire_gathers(gb, ib):
            for j in range(N_SUB):
                pltpu.async_copy(
                    table_hbm.at[idx_v.at[ib, j]],
                    rows_v.at[gb, pl.ds(j * SUB, SUB)],
                    sem_g,
                )

        def drain_gathers(gb):
            for j in range(N_SUB):
                pltpu.make_async_copy(
                    table_hbm.at[idx_v.at[0, j]],
                    rows_v.at[gb, pl.ds(j * SUB, SUB)],
                    sem_g,
                ).wait()

        def stage_idx(g, ib, async_=True):
            if async_:
                pltpu.async_copy(ids_hbm.at[g0 + g], idx_v.at[ib], sem_i)
            else:
                pltpu.sync_copy(ids_hbm.at[g0 + g], idx_v.at[ib])

        def drain_idx():
            pltpu.make_async_copy(ids_hbm.at[g0], idx_v.at[0], sem_i).wait()

        def start_write(g, gb):
            pltpu.async_copy(
                rows_v.at[gb], out_hbm.at[pl.ds(base + g * G, G)], sem_w
            )

        def drain_write(gb):
            pltpu.make_async_copy(
                rows_v.at[gb], out_hbm.at[pl.ds(base, G)], sem_w
            ).wait()

        stage_idx(0, 0, async_=False)
        fire_gathers(0, 0)
        stage_idx(1, 1)

        def loop_body(g, carry):
            b = lax.rem(g, NBUF)
            nb = lax.rem(g + 1, NBUF)

            @pl.when(g + 1 < n_gran)
            def _fire_next():
                drain_idx()

                @pl.when(g >= 2)
                def _reclaim():
                    drain_write(nb)

                fire_gathers(nb, nb)

            drain_gathers(b)

            @pl.when(g + 2 < n_gran)
            def _prefetch_idx():
                stage_idx(g + 2, lax.rem(g + 2, NBUF))

            start_write(g, b)
            return carry

        lax.fori_loop(0, n_gran, loop_body, 0)

        for t in range(min(NBUF, n_gran)):
            drain_write(t)

    return gather_kernel


def kernel(token_ids, weights):
    B0, S = token_ids.shape
    V, D = weights.shape
    B = B0 * S
    ids = token_ids.reshape(B // G, N_SUB, SUB).astype(jnp.int32)
    out = _build(B, V, D)(ids, weights)
    return out.reshape(B0, S, D)
